# baseline (speedup 1.0000x reference)
import jax
import jax.numpy as jnp
from jax.experimental import pallas as pl

_N = 4096
_D = 64
_BM = 256


def _matmul_kernel(adj_ref, emb_ref, out_ref):
    out_ref[...] = jnp.dot(
        adj_ref[...], emb_ref[...], preferred_element_type=jnp.float32
    )


def kernel(adj, embeds):
    return pl.pallas_call(
        _matmul_kernel,
        grid=(_N // _BM,),
        in_specs=[
            pl.BlockSpec((_BM, _N), lambda i: (i, 0)),
            pl.BlockSpec((_N, _D), lambda i: (0, 0)),
        ],
        out_specs=pl.BlockSpec((_BM, _D), lambda i: (i, 0)),
        out_shape=jax.ShapeDtypeStruct((_N, _D), jnp.float32),
    )(adj, embeds)


# speedup vs baseline: 1.1237x; 1.1237x over previous
import jax
import jax.numpy as jnp
from jax.experimental import pallas as pl

_N = 4096
_D = 64
_BM = 512


def _matmul_kernel(adj_ref, emb_ref, out_ref):
    out_ref[...] = jnp.dot(
        adj_ref[...], emb_ref[...], preferred_element_type=jnp.float32
    )


def kernel(adj, embeds):
    return pl.pallas_call(
        _matmul_kernel,
        grid=(_N // _BM,),
        in_specs=[
            pl.BlockSpec((_BM, _N), lambda i: (i, 0)),
            pl.BlockSpec((_N, _D), lambda i: (0, 0)),
        ],
        out_specs=pl.BlockSpec((_BM, _D), lambda i: (i, 0)),
        out_shape=jax.ShapeDtypeStruct((_N, _D), jnp.float32),
    )(adj, embeds)
